# baseline (speedup 1.0000x reference)
import functools

import jax
import jax.numpy as jnp
from jax.experimental import pallas as pl

_F32 = jnp.float32
_BF16 = jnp.bfloat16


def _mm(a, b):
    return jax.lax.dot_general(
        a.astype(_BF16), b, (((1,), (0,)), ((), ())),
        preferred_element_type=_F32)


def _enc1_body(x_ref, w_ref, b_ref, o_ref, *, RC, W_out):
    r = pl.program_id(1) * RC
    parts = []
    for kh in range(4):
        ty, sy = divmod(kh, 2)
        for kw in range(4):
            tx, sx = divmod(kw, 2)
            c0 = (sy * 2 + sx) * 3
            xs = x_ref[0, pl.ds(r + ty, RC), pl.ds(tx, W_out), c0:c0 + 3]
            parts.append(xs.reshape(RC * W_out, 3))
    xcat = jnp.concatenate(parts, axis=1)
    acc = _mm(xcat, w_ref[...]) + b_ref[...]
    o_ref[0] = jnp.maximum(acc, 0.0).reshape(RC, W_out, 128)


def _enc1(x_pad, w_flat, bias, W_out, RC):
    B, Hp, Wp, C_in = x_pad.shape
    body = functools.partial(_enc1_body, RC=RC, W_out=W_out)
    return pl.pallas_call(
        body,
        grid=(B, W_out // RC),
        in_specs=[
            pl.BlockSpec((1, Hp, Wp, C_in), lambda b, r: (b, 0, 0, 0)),
            pl.BlockSpec((48, 128), lambda b, r: (0, 0)),
            pl.BlockSpec((1, 128), lambda b, r: (0, 0)),
        ],
        out_specs=pl.BlockSpec((1, RC, W_out, 128), lambda b, r: (b, r, 0, 0)),
        out_shape=jax.ShapeDtypeStruct((B, W_out, W_out, 128), _F32),
    )(x_pad, w_flat, bias)


def _enc2_body(x_ref, w_ref, b_ref, o_ref, *, RC, W_out):
    r = pl.program_id(1) * RC
    acc = jnp.zeros((RC * W_out, 256), _F32)
    for kh in range(4):
        ty, sy = divmod(kh, 2)
        for txx in range(2):
            xs = x_ref[0, pl.ds(r + ty, RC), pl.ds(txx, W_out),
                       sy * 256:(sy + 1) * 256]
            acc = acc + _mm(xs.reshape(RC * W_out, 256), w_ref[kh * 2 + txx])
    acc = acc + b_ref[...]
    o_ref[0] = jnp.maximum(acc, 0.0).reshape(RC, W_out, 256)


def _enc2(x_pad, w_pass, bias, W_out, RC):
    B, Hp, Wp, C_in = x_pad.shape
    body = functools.partial(_enc2_body, RC=RC, W_out=W_out)
    return pl.pallas_call(
        body,
        grid=(B, W_out // RC),
        in_specs=[
            pl.BlockSpec((1, Hp, Wp, C_in), lambda b, r: (b, 0, 0, 0)),
            pl.BlockSpec((8, 256, 256), lambda b, r: (0, 0, 0)),
            pl.BlockSpec((1, 256), lambda b, r: (0, 0)),
        ],
        out_specs=pl.BlockSpec((1, RC, W_out, 256), lambda b, r: (b, r, 0, 0)),
        out_shape=jax.ShapeDtypeStruct((B, W_out, W_out, 256), _F32),
    )(x_pad, w_pass, bias)


def _conv_body(x_ref, w_ref, b_ref, o_ref, *, KH, KW, W_out, RC, C_in, C_out,
               relu):
    r = pl.program_id(1) * RC
    acc = jnp.broadcast_to(b_ref[...], (RC * W_out, C_out))
    for dy in range(KH):
        for dx in range(KW):
            xs = x_ref[0, pl.ds(r + dy, RC), pl.ds(dx, W_out), :]
            xs = xs.reshape(RC * W_out, C_in)
            acc = acc + _mm(xs, w_ref[dy, dx])
    if relu:
        acc = jnp.maximum(acc, 0.0)
    o_ref[0] = acc.reshape(RC, W_out, C_out)


def _tap_conv(x_pad, w_taps, bias, W_out, RC, relu):
    B, Hp, Wp, C_in = x_pad.shape
    KH, KW, _, C_out = w_taps.shape
    H_out = Hp - KH + 1
    body = functools.partial(_conv_body, KH=KH, KW=KW, W_out=W_out, RC=RC,
                             C_in=C_in, C_out=C_out, relu=relu)
    return pl.pallas_call(
        body,
        grid=(B, H_out // RC),
        in_specs=[
            pl.BlockSpec((1, Hp, Wp, C_in), lambda b, r: (b, 0, 0, 0)),
            pl.BlockSpec((KH, KW, C_in, C_out), lambda b, r: (0, 0, 0, 0)),
            pl.BlockSpec((1, C_out), lambda b, r: (0, 0)),
        ],
        out_specs=pl.BlockSpec((1, RC, W_out, C_out), lambda b, r: (b, r, 0, 0)),
        out_shape=jax.ShapeDtypeStruct((B, H_out, W_out, C_out), _F32),
    )(x_pad, w_taps, bias)


def _phase_body(x_ref, w_ref, b_ref, o00, o01, o10, o11, *, W_out, RC, C_in,
                C_out, relu):
    r = pl.program_id(1) * RC
    outs = ((o00, o01), (o10, o11))
    for sy in range(2):
        for sx in range(2):
            acc = jnp.broadcast_to(b_ref[...], (RC * W_out, C_out))
            for u in range(2):
                for v in range(2):
                    xs = x_ref[0, pl.ds(r + sy + u, RC), pl.ds(sx + v, W_out), :]
                    xs = xs.reshape(RC * W_out, C_in)
                    acc = acc + _mm(xs, w_ref[sy, sx, u, v])
            if relu:
                acc = jnp.maximum(acc, 0.0)
            outs[sy][sx][0] = acc.reshape(RC, W_out, C_out)


def _phase_conv(x_pad, w_phase, bias, W_out, RC, relu):
    B, Hp, Wp, C_in = x_pad.shape
    C_out = w_phase.shape[-1]
    body = functools.partial(_phase_body, W_out=W_out, RC=RC, C_in=C_in,
                             C_out=C_out, relu=relu)
    shp = jax.ShapeDtypeStruct((B, W_out, W_out, C_out), _F32)
    return pl.pallas_call(
        body,
        grid=(B, W_out // RC),
        in_specs=[
            pl.BlockSpec((1, Hp, Wp, C_in), lambda b, r: (b, 0, 0, 0)),
            pl.BlockSpec((2, 2, 2, 2, C_in, C_out),
                         lambda b, r: (0, 0, 0, 0, 0, 0)),
            pl.BlockSpec((1, C_out), lambda b, r: (0, 0)),
        ],
        out_specs=[pl.BlockSpec((1, RC, W_out, C_out),
                                lambda b, r: (b, r, 0, 0))] * 4,
        out_shape=[shp] * 4,
    )(x_pad, w_phase, bias)


def _fold_phase_weights(w):
    a0 = jnp.stack([w[0], w[1] + w[2]])
    a1 = jnp.stack([w[0] + w[1], w[2]])
    ws = []
    for a in (a0, a1):
        b0 = jnp.stack([a[:, 0], a[:, 1] + a[:, 2]], axis=1)
        b1 = jnp.stack([a[:, 0] + a[:, 1], a[:, 2]], axis=1)
        ws.append(jnp.stack([b0, b1]))
    return jnp.stack(ws)


def _interleave_phases(p, C):
    y = jnp.stack([jnp.stack(row) for row in p])
    B, n = y.shape[2], y.shape[3]
    y = y.transpose(2, 3, 0, 4, 1, 5)
    return y.reshape(B, 2 * n, 2 * n, C)


def _enc3_vq_body(x_ref, w_ref, b_ref, cbt_ref, cb_ref, cb3_ref, q_ref, i_ref,
                  l_ref, *, RC, W_out, C, K):
    r = pl.program_id(1) * RC
    acc = jnp.zeros((RC * W_out, C), _F32)
    for dy in range(3):
        for dx in range(3):
            xs = x_ref[0, pl.ds(r + dy, RC), pl.ds(dx, W_out), :]
            xs = xs.reshape(RC * W_out, C)
            acc = acc + _mm(xs, w_ref[dy, dx])
    z = acc + b_ref[...]
    csq = jnp.sum(cb_ref[...] ** 2, axis=1)
    lacc = jnp.float32(0.0)
    for i in range(RC):
        zr = z[i * W_out:(i + 1) * W_out]
        zsq = jnp.sum(zr ** 2, axis=1, keepdims=True)
        d = zsq - 2.0 * _mm(zr, cbt_ref[...]) + csq[None, :]
        idx = jnp.argmin(d, axis=1)
        i_ref[0, i, :] = idx
        onehot = (jax.lax.broadcasted_iota(jnp.int32, (W_out, K), 1)
                  == idx[:, None]).astype(_BF16)
        q = (_mm(onehot, cb3_ref[0])
             + (_mm(onehot, cb3_ref[1]) + _mm(onehot, cb3_ref[2])))
        lacc = lacc + jnp.sum((q - zr) ** 2)
        q_ref[0, i] = (zr + (q - zr)).reshape(W_out, C)
    l_ref[0, 0, 0, :] = jnp.full((128,), lacc, _F32)


def _enc3_vq(x_pad, w_taps, bias, cbt, cb, cb3, W_out, RC):
    B, Hp, Wp, C = x_pad.shape
    K = cb.shape[0]
    body = functools.partial(_enc3_vq_body, RC=RC, W_out=W_out, C=C, K=K)
    nblk = W_out // RC
    return pl.pallas_call(
        body,
        grid=(B, nblk),
        in_specs=[
            pl.BlockSpec((1, Hp, Wp, C), lambda b, r: (b, 0, 0, 0)),
            pl.BlockSpec((3, 3, C, C), lambda b, r: (0, 0, 0, 0)),
            pl.BlockSpec((1, C), lambda b, r: (0, 0)),
            pl.BlockSpec((C, K), lambda b, r: (0, 0)),
            pl.BlockSpec((K, C), lambda b, r: (0, 0)),
            pl.BlockSpec((3, K, C), lambda b, r: (0, 0, 0)),
        ],
        out_specs=[
            pl.BlockSpec((1, RC, W_out, C), lambda b, r: (b, r, 0, 0)),
            pl.BlockSpec((1, RC, W_out), lambda b, r: (b, r, 0)),
            pl.BlockSpec((1, 1, 1, 128), lambda b, r: (b, r, 0, 0)),
        ],
        out_shape=[
            jax.ShapeDtypeStruct((B, W_out, W_out, C), _F32),
            jax.ShapeDtypeStruct((B, W_out, W_out), jnp.int32),
            jax.ShapeDtypeStruct((B, nblk, 1, 128), _F32),
        ],
    )(x_pad, w_taps, bias, cbt, cb, cb3)


def _pad_hw(x, p=1):
    return jnp.pad(x, ((0, 0), (p, p), (p, p), (0, 0)))


def _s2d(x):
    B, H, W, C = x.shape
    x = x.reshape(B, H // 2, 2, W // 2, 2, C)
    x = x.transpose(0, 1, 3, 2, 4, 5)
    return x.reshape(B, H // 2, W // 2, 4 * C)


def _s2d_weights(w):
    O, C = w.shape[0], w.shape[1]
    w = w.reshape(O, C, 2, 2, 2, 2)
    w = w.transpose(2, 4, 3, 5, 1, 0)
    return w.reshape(2, 2, 4 * C, O)


def kernel(x, enc_w1, enc_b1, enc_w2, enc_b2, enc_w3, enc_b3, codebook,
           dec_w1, dec_b1, dec_w2, dec_b2, dec_w3, dec_b3):
    B = x.shape[0]
    x1 = _s2d(_pad_hw(jnp.transpose(x, (0, 2, 3, 1))))
    w1f = jnp.transpose(enc_w1, (2, 3, 1, 0)).reshape(48, 128)
    h1 = _enc1(x1, w1f.astype(_BF16), enc_b1[None], 112, 8)
    x2 = _s2d(_pad_hw(h1))
    w2p = jnp.transpose(enc_w2, (2, 3, 1, 0)).reshape(4, 2, 256, 256)
    w2p = w2p.reshape(8, 256, 256)
    h2 = _enc2(x2, w2p.astype(_BF16), enc_b2[None], 56, 8)
    w3 = jnp.transpose(enc_w3, (2, 3, 1, 0)).astype(_BF16)
    cb_hi = codebook.astype(_BF16)
    r1 = codebook - cb_hi.astype(_F32)
    cb_mid = r1.astype(_BF16)
    cb_lo = (r1 - cb_mid.astype(_F32)).astype(_BF16)
    cb3 = jnp.stack([cb_hi, cb_mid, cb_lo])
    quant, indice, lpart = _enc3_vq(_pad_hw(h2), w3, enc_b3[None],
                                    codebook.T.astype(_BF16), codebook,
                                    cb3, 56, 8)
    n_el = quant.shape[0] * quant.shape[1] * quant.shape[2] * quant.shape[3]
    codebook_loss = jnp.sum(lpart[:, :, 0, 0]) / n_el
    commit_loss = 0.25 * codebook_loss
    dw1 = jnp.transpose(dec_w1, (2, 3, 1, 0)).astype(_BF16)
    g1 = _tap_conv(_pad_hw(quant), dw1, dec_b1[None], 56, 8, True)
    dw2 = _fold_phase_weights(jnp.transpose(dec_w2, (2, 3, 1, 0)))
    p2 = _phase_conv(_pad_hw(g1), dw2.astype(_BF16), dec_b2[None], 56, 8, True)
    g2 = _interleave_phases(((p2[0], p2[1]), (p2[2], p2[3])), 128)
    dw3 = jnp.transpose(dec_w3, (2, 3, 1, 0))
    dw3 = jnp.pad(dw3, ((0, 0), (0, 0), (0, 0), (0, 5)))
    db3 = jnp.pad(dec_b3, (0, 5))
    p3 = _phase_conv(_pad_hw(g2), _fold_phase_weights(dw3).astype(_BF16),
                     db3[None], 112, 8, False)
    xr = _interleave_phases(((p3[0], p3[1]), (p3[2], p3[3])), 8)[..., :3]
    x_recon = jnp.transpose(xr, (0, 3, 1, 2))
    return (x_recon, indice, codebook_loss, commit_loss)


# speedup vs baseline: 1.2432x; 1.2432x over previous
import functools

import jax
import jax.numpy as jnp
from jax.experimental import pallas as pl
from jax.experimental.pallas import tpu as pltpu

_F32 = jnp.float32
_BF16 = jnp.bfloat16


def _mm(a, b):
    return jax.lax.dot_general(
        a.astype(_BF16), b, (((1,), (0,)), ((), ())),
        preferred_element_type=_F32)


def _enc1_body(x_ref, w_ref, b_ref, o_ref, *, RC, W_out):
    r = pl.program_id(1) * RC
    parts = []
    for kh in range(4):
        ty, sy = divmod(kh, 2)
        for kw in range(4):
            tx, sx = divmod(kw, 2)
            c0 = (sy * 2 + sx) * 3
            xs = x_ref[0, pl.ds(r + ty, RC), pl.ds(tx, W_out), c0:c0 + 3]
            parts.append(xs.reshape(RC * W_out, 3))
    xcat = jnp.concatenate(parts, axis=1)
    acc = _mm(xcat, w_ref[...]) + b_ref[...]
    o_ref[0] = jnp.maximum(acc, 0.0).reshape(RC, W_out, 128).astype(_BF16)


def _enc1(x_pad, w_flat, bias, W_out, RC):
    B, Hp, Wp, C_in = x_pad.shape
    body = functools.partial(_enc1_body, RC=RC, W_out=W_out)
    return pl.pallas_call(
        body,
        grid=(B, W_out // RC),
        in_specs=[
            pl.BlockSpec((1, Hp, Wp, C_in), lambda b, r: (b, 0, 0, 0)),
            pl.BlockSpec((48, 128), lambda b, r: (0, 0)),
            pl.BlockSpec((1, 128), lambda b, r: (0, 0)),
        ],
        out_specs=pl.BlockSpec((1, RC, W_out, 128), lambda b, r: (b, r, 0, 0)),
        out_shape=jax.ShapeDtypeStruct((B, W_out, W_out, 128), _BF16),
    )(x_pad, w_flat, bias)


def _mega_body(x2_ref, w2_ref, b2_ref, w3_ref, b3_ref, cbt_ref, cb_ref,
               cb3_ref, dw1_ref, db1_ref, dw2_ref, db2_ref,
               i_ref, l_ref, p00, p01, p10, p11, sh2, sz, sq, sg1):
    W = 56

    @pl.when(pl.program_id(0) == 0)
    def _zero_borders():
        for s in (sh2, sq, sg1):
            s[0:1, :, :] = jnp.zeros((1, 58, 256), _BF16)
            s[57:58, :, :] = jnp.zeros((1, 58, 256), _BF16)
            s[:, 0:1, :] = jnp.zeros((58, 1, 256), _BF16)
            s[:, 57:58, :] = jnp.zeros((58, 1, 256), _BF16)

    for r0 in range(0, W, 8):
        acc = jnp.zeros((8 * W, 256), _F32)
        for kh in range(4):
            ty, sy = divmod(kh, 2)
            for txx in range(2):
                xs = x2_ref[0, pl.ds(r0 + ty, 8), pl.ds(txx, W),
                            sy * 256:(sy + 1) * 256]
                acc = acc + _mm(xs.reshape(8 * W, 256), w2_ref[kh * 2 + txx])
        h = jnp.maximum(acc + b2_ref[...], 0.0)
        sh2[pl.ds(r0 + 1, 8), 1:57, :] = h.reshape(8, W, 256).astype(_BF16)

    for r0 in range(0, W, 8):
        acc = jnp.zeros((8 * W, 256), _F32)
        for dy in range(3):
            for dx in range(3):
                xs = sh2[pl.ds(r0 + dy, 8), pl.ds(dx, W), :]
                acc = acc + _mm(xs.reshape(8 * W, 256), w3_ref[dy, dx])
        sz[pl.ds(r0 * W, 8 * W), :] = acc + b3_ref[...]

    csq = jnp.sum(cb_ref[...] ** 2, axis=1)

    def vq_step(i, lacc):
        zr = sz[pl.ds(i * 112, 112), :]
        zsq = jnp.sum(zr ** 2, axis=1, keepdims=True)
        d = zsq - 2.0 * _mm(zr, cbt_ref[...]) + csq[None, :]
        idx = jnp.argmin(d, axis=1)
        i_ref[0, pl.ds(i * 2, 2), :] = idx.reshape(2, W)
        oh = (jax.lax.broadcasted_iota(jnp.int32, (112, 1024), 1)
              == idx[:, None]).astype(_BF16)
        q = _mm(oh, cb3_ref[0]) + (_mm(oh, cb3_ref[1]) + _mm(oh, cb3_ref[2]))
        sq[pl.ds(1 + i * 2, 2), 1:57, :] = (
            (zr + (q - zr)).reshape(2, W, 256).astype(_BF16))
        return lacc + jnp.sum((q - zr) ** 2)

    lacc = jax.lax.fori_loop(0, 28, vq_step, jnp.float32(0.0))
    l_ref[0, 0, 0, :] = jnp.full((128,), lacc, _F32)

    for r0 in range(0, W, 8):
        acc = jnp.zeros((8 * W, 256), _F32)
        for dy in range(3):
            for dx in range(3):
                xs = sq[pl.ds(r0 + dy, 8), pl.ds(dx, W), :]
                acc = acc + _mm(xs.reshape(8 * W, 256), dw1_ref[dy, dx])
        g = jnp.maximum(acc + db1_ref[...], 0.0)
        sg1[pl.ds(r0 + 1, 8), 1:57, :] = g.reshape(8, W, 256).astype(_BF16)

    outs = ((p00, p01), (p10, p11))
    for r0 in range(0, W, 8):
        for sy in range(2):
            for sx in range(2):
                acc = jnp.zeros((8 * W, 128), _F32)
                for u in range(2):
                    for v in range(2):
                        xs = sg1[pl.ds(r0 + sy + u, 8), pl.ds(sx + v, W), :]
                        acc = acc + _mm(xs.reshape(8 * W, 256),
                                        dw2_ref[sy, sx, u, v])
                g = jnp.maximum(acc + db2_ref[...], 0.0)
                outs[sy][sx][0, pl.ds(r0, 8)] = (
                    g.reshape(8, W, 128).astype(_BF16))


def _mega(x2, w2p, b2, w3, b3, cbt, cb, cb3, dw1, db1, dw2p, db2):
    B = x2.shape[0]
    shp_p = jax.ShapeDtypeStruct((B, 56, 56, 128), _BF16)
    return pl.pallas_call(
        _mega_body,
        grid=(B,),
        in_specs=[
            pl.BlockSpec((1, 57, 57, 512), lambda b: (b, 0, 0, 0)),
            pl.BlockSpec((8, 256, 256), lambda b: (0, 0, 0)),
            pl.BlockSpec((1, 256), lambda b: (0, 0)),
            pl.BlockSpec((3, 3, 256, 256), lambda b: (0, 0, 0, 0)),
            pl.BlockSpec((1, 256), lambda b: (0, 0)),
            pl.BlockSpec((256, 1024), lambda b: (0, 0)),
            pl.BlockSpec((1024, 256), lambda b: (0, 0)),
            pl.BlockSpec((3, 1024, 256), lambda b: (0, 0, 0)),
            pl.BlockSpec((3, 3, 256, 256), lambda b: (0, 0, 0, 0)),
            pl.BlockSpec((1, 256), lambda b: (0, 0)),
            pl.BlockSpec((2, 2, 2, 2, 256, 128), lambda b: (0, 0, 0, 0, 0, 0)),
            pl.BlockSpec((1, 128), lambda b: (0, 0)),
        ],
        out_specs=[
            pl.BlockSpec((1, 56, 56), lambda b: (b, 0, 0)),
            pl.BlockSpec((1, 1, 1, 128), lambda b: (b, 0, 0, 0)),
            pl.BlockSpec((1, 56, 56, 128), lambda b: (b, 0, 0, 0)),
            pl.BlockSpec((1, 56, 56, 128), lambda b: (b, 0, 0, 0)),
            pl.BlockSpec((1, 56, 56, 128), lambda b: (b, 0, 0, 0)),
            pl.BlockSpec((1, 56, 56, 128), lambda b: (b, 0, 0, 0)),
        ],
        out_shape=[
            jax.ShapeDtypeStruct((B, 56, 56), jnp.int32),
            jax.ShapeDtypeStruct((B, 1, 1, 128), _F32),
            shp_p, shp_p, shp_p, shp_p,
        ],
        scratch_shapes=[
            pltpu.VMEM((58, 58, 256), _BF16),
            pltpu.VMEM((3136, 256), _F32),
            pltpu.VMEM((58, 58, 256), _BF16),
            pltpu.VMEM((58, 58, 256), _BF16),
        ],
    )(x2, w2p, b2, w3, b3, cbt, cb, cb3, dw1, db1, dw2p, db2)


def _phase_body(x_ref, w_ref, b_ref, o00, o01, o10, o11, *, W_out, RC, C_in,
                C_out):
    r = pl.program_id(1) * RC
    outs = ((o00, o01), (o10, o11))
    for sy in range(2):
        for sx in range(2):
            acc = jnp.zeros((RC * W_out, C_out), _F32)
            for u in range(2):
                for v in range(2):
                    xs = x_ref[0, pl.ds(r + sy + u, RC), pl.ds(sx + v, W_out), :]
                    xs = xs.reshape(RC * W_out, C_in)
                    acc = acc + _mm(xs, w_ref[sy, sx, u, v])
            acc = acc + b_ref[...]
            outs[sy][sx][0] = acc.reshape(RC, W_out, C_out)


def _phase_conv(x_pad, w_phase, bias, W_out, RC):
    B, Hp, Wp, C_in = x_pad.shape
    C_out = w_phase.shape[-1]
    body = functools.partial(_phase_body, W_out=W_out, RC=RC, C_in=C_in,
                             C_out=C_out)
    shp = jax.ShapeDtypeStruct((B, W_out, W_out, C_out), _F32)
    return pl.pallas_call(
        body,
        grid=(B, W_out // RC),
        in_specs=[
            pl.BlockSpec((1, Hp, Wp, C_in), lambda b, r: (b, 0, 0, 0)),
            pl.BlockSpec((2, 2, 2, 2, C_in, C_out),
                         lambda b, r: (0, 0, 0, 0, 0, 0)),
            pl.BlockSpec((1, C_out), lambda b, r: (0, 0)),
        ],
        out_specs=[pl.BlockSpec((1, RC, W_out, C_out),
                                lambda b, r: (b, r, 0, 0))] * 4,
        out_shape=[shp] * 4,
    )(x_pad, w_phase, bias)


def _fold_phase_weights(w):
    a0 = jnp.stack([w[0], w[1] + w[2]])
    a1 = jnp.stack([w[0] + w[1], w[2]])
    ws = []
    for a in (a0, a1):
        b0 = jnp.stack([a[:, 0], a[:, 1] + a[:, 2]], axis=1)
        b1 = jnp.stack([a[:, 0] + a[:, 1], a[:, 2]], axis=1)
        ws.append(jnp.stack([b0, b1]))
    return jnp.stack(ws)


def _interleave_phases(p, C):
    y = jnp.stack([jnp.stack(row) for row in p])
    B, n = y.shape[2], y.shape[3]
    y = y.transpose(2, 3, 0, 4, 1, 5)
    return y.reshape(B, 2 * n, 2 * n, C)


def _pad_hw(x, p=1):
    return jnp.pad(x, ((0, 0), (p, p), (p, p), (0, 0)))


def _s2d(x):
    B, H, W, C = x.shape
    x = x.reshape(B, H // 2, 2, W // 2, 2, C)
    x = x.transpose(0, 1, 3, 2, 4, 5)
    return x.reshape(B, H // 2, W // 2, 4 * C)


def kernel(x, enc_w1, enc_b1, enc_w2, enc_b2, enc_w3, enc_b3, codebook,
           dec_w1, dec_b1, dec_w2, dec_b2, dec_w3, dec_b3):
    B = x.shape[0]
    x1 = _s2d(_pad_hw(jnp.transpose(x, (0, 2, 3, 1)))).astype(_BF16)
    w1f = jnp.transpose(enc_w1, (2, 3, 1, 0)).reshape(48, 128)
    h1 = _enc1(x1, w1f.astype(_BF16), enc_b1[None], 112, 8)
    x2 = _s2d(_pad_hw(h1))
    w2p = jnp.transpose(enc_w2, (2, 3, 1, 0)).reshape(8, 256, 256)
    w3 = jnp.transpose(enc_w3, (2, 3, 1, 0))
    cb_hi = codebook.astype(_BF16)
    r1 = codebook - cb_hi.astype(_F32)
    cb_mid = r1.astype(_BF16)
    cb_lo = (r1 - cb_mid.astype(_F32)).astype(_BF16)
    cb3 = jnp.stack([cb_hi, cb_mid, cb_lo])
    dw1 = jnp.transpose(dec_w1, (2, 3, 1, 0))
    dw2p = _fold_phase_weights(jnp.transpose(dec_w2, (2, 3, 1, 0)))
    outs = _mega(x2, w2p.astype(_BF16), enc_b2[None], w3.astype(_BF16),
                 enc_b3[None], codebook.T.astype(_BF16), codebook, cb3,
                 dw1.astype(_BF16), dec_b1[None], dw2p.astype(_BF16),
                 dec_b2[None])
    indice, lpart, q00, q01, q10, q11 = outs
    n_el = B * 56 * 56 * 256
    codebook_loss = jnp.sum(lpart[:, 0, 0, 0]) / n_el
    commit_loss = 0.25 * codebook_loss
    g2 = _interleave_phases(((q00, q01), (q10, q11)), 128)
    dw3 = jnp.transpose(dec_w3, (2, 3, 1, 0))
    dw3 = jnp.pad(dw3, ((0, 0), (0, 0), (0, 0), (0, 5)))
    db3 = jnp.pad(dec_b3, (0, 5))
    p3 = _phase_conv(_pad_hw(g2), _fold_phase_weights(dw3).astype(_BF16),
                     db3[None], 112, 8)
    xr = _interleave_phases(((p3[0], p3[1]), (p3[2], p3[3])), 8)[..., :3]
    x_recon = jnp.transpose(xr, (0, 3, 1, 2))
    return (x_recon, indice, codebook_loss, commit_loss)
